# baseline (speedup 1.0000x reference)
import jax
import jax.numpy as jnp
from jax import lax
from jax.experimental import pallas as pl
from jax.experimental.pallas import tpu as pltpu
from jax.experimental.pallas import tpu_sc as plsc

NUM_ROWS_TABLE = 1_000_000
EMBED_DIM = 32
BATCH_SIZE = 16384

NC = 2
NS = 16
LANES = 16
NW = NC * NS
B_PER_W = BATCH_SIZE // NW
GROUPS = B_PER_W // LANES


def _body(uid_hbm, iid_hbm, uemb_hbm, iemb_hbm, ub_hbm, ib_hbm, mean_hbm,
          out_hbm, uid_v, iid_v, mean_v, u_buf0, u_buf1, i_buf0, i_buf1,
          ub_b0, ub_b1, ib_b0, ib_b1, out_v,
          emb_sem0, emb_sem1, bias_sem0, bias_sem1):
    wid = lax.axis_index("s") * NC + lax.axis_index("c")
    base = wid * B_PER_W

    pltpu.sync_copy(uid_hbm.at[pl.ds(base, B_PER_W)], uid_v)
    pltpu.sync_copy(iid_hbm.at[pl.ds(base, B_PER_W)], iid_v)
    pltpu.sync_copy(mean_hbm, mean_v)

    u_bufs = (u_buf0, u_buf1)
    i_bufs = (i_buf0, i_buf1)
    ub_bufs = (ub_b0, ub_b1)
    ib_bufs = (ib_b0, ib_b1)
    esems = (emb_sem0, emb_sem1)
    bsems = (bias_sem0, bias_sem1)

    def fire_group(g, parity):
        uids = uid_v[pl.ds(g * LANES, LANES)]
        iids = iid_v[pl.ds(g * LANES, LANES)]
        for r in range(LANES):
            uid = uids[r]
            iid = iids[r]
            pltpu.async_copy(uemb_hbm.at[pl.ds(uid, 1), :],
                             u_bufs[parity].at[pl.ds(r, 1), :], esems[parity])
            pltpu.async_copy(iemb_hbm.at[pl.ds(iid, 1), :],
                             i_bufs[parity].at[pl.ds(r, 1), :], esems[parity])
            pltpu.async_copy(ub_hbm.at[pl.ds(uid, 1), :],
                             ub_bufs[parity].at[pl.ds(r, 1), :], bsems[parity])
            pltpu.async_copy(ib_hbm.at[pl.ds(iid, 1), :],
                             ib_bufs[parity].at[pl.ds(r, 1), :], bsems[parity])

    def drain_group(parity):
        pltpu.make_async_copy(uemb_hbm.at[pl.ds(0, LANES), :],
                              u_bufs[parity], esems[parity]).wait()
        pltpu.make_async_copy(iemb_hbm.at[pl.ds(0, LANES), :],
                              i_bufs[parity], esems[parity]).wait()
        pltpu.make_async_copy(ub_hbm.at[pl.ds(0, LANES), :],
                              ub_bufs[parity], bsems[parity]).wait()
        pltpu.make_async_copy(ib_hbm.at[pl.ds(0, LANES), :],
                              ib_bufs[parity], bsems[parity]).wait()

    H = EMBED_DIM // 2
    lane = lax.iota(jnp.int32, LANES)
    zeros16 = jnp.zeros((LANES,), jnp.int32)
    zf16 = jnp.zeros((LANES,), jnp.float32)
    mean16 = mean_v[...]

    def compute_group(g, parity):
        u_buf, i_buf = u_bufs[parity], i_bufs[parity]
        acc = zf16
        for r in range(LANES):
            u0 = u_buf[r, pl.ds(0, H)]
            u1 = u_buf[r, pl.ds(H, H)]
            i0 = i_buf[r, pl.ds(0, H)]
            i1 = i_buf[r, pl.ds(H, H)]
            p = u0 * i0 + u1 * i1
            acc = jnp.where(lane == r, jnp.sum(p), acc)
        ubx = plsc.load_gather(ub_bufs[parity], [lane, zeros16])
        ibx = plsc.load_gather(ib_bufs[parity], [lane, zeros16])
        out_v[pl.ds(g * LANES, LANES)] = acc + ubx + ibx + mean16

    fire_group(0, 0)

    def pair(g2, carry):
        g = 2 * g2
        drain_group(0)
        fire_group(g + 1, 1)
        compute_group(g, 0)
        drain_group(1)

        @pl.when(g + 2 < GROUPS)
        def _():
            fire_group(g + 2, 0)

        compute_group(g + 1, 1)
        return carry

    lax.fori_loop(0, GROUPS // 2, pair, 0)

    pltpu.sync_copy(out_v, out_hbm.at[pl.ds(base, B_PER_W)])


@jax.jit
def _run(u_id, i_id, user_emb, item_emb, user_bias, item_bias, mean16):
    mesh = plsc.VectorSubcoreMesh(core_axis_name="c", subcore_axis_name="s")
    call = pl.kernel(
        _body,
        out_type=jax.ShapeDtypeStruct((BATCH_SIZE,), jnp.float32),
        mesh=mesh,
        compiler_params=pltpu.CompilerParams(
            needs_layout_passes=False, use_tc_tiling_on_sc=True),
        scratch_types=[
            pltpu.VMEM((B_PER_W,), jnp.int32),
            pltpu.VMEM((B_PER_W,), jnp.int32),
            pltpu.VMEM((LANES,), jnp.float32),
            pltpu.VMEM((LANES, EMBED_DIM), jnp.float32),
            pltpu.VMEM((LANES, EMBED_DIM), jnp.float32),
            pltpu.VMEM((LANES, EMBED_DIM), jnp.float32),
            pltpu.VMEM((LANES, EMBED_DIM), jnp.float32),
            pltpu.VMEM((LANES, 1), jnp.float32),
            pltpu.VMEM((LANES, 1), jnp.float32),
            pltpu.VMEM((LANES, 1), jnp.float32),
            pltpu.VMEM((LANES, 1), jnp.float32),
            pltpu.VMEM((B_PER_W,), jnp.float32),
            pltpu.SemaphoreType.DMA,
            pltpu.SemaphoreType.DMA,
            pltpu.SemaphoreType.DMA,
            pltpu.SemaphoreType.DMA,
        ],
    )
    return call(u_id, i_id, user_emb, item_emb, user_bias, item_bias, mean16)


def kernel(u_id, i_id, user_emb, item_emb, user_bias, item_bias, mean):
    mean16 = jnp.broadcast_to(mean.astype(jnp.float32).reshape(()), (LANES,))
    return _run(u_id.astype(jnp.int32), i_id.astype(jnp.int32),
                user_emb, item_emb, user_bias, item_bias, mean16)


# speedup vs baseline: 1.1736x; 1.1736x over previous
import jax
import jax.numpy as jnp
from jax import lax
from jax.experimental import pallas as pl
from jax.experimental.pallas import tpu as pltpu
from jax.experimental.pallas import tpu_sc as plsc

NUM_ROWS_TABLE = 1_000_000
EMBED_DIM = 32
BATCH_SIZE = 16384

NC = 2
NS = 16
LANES = 16
NW = NC * NS
B_PER_W = BATCH_SIZE // NW
IDX_CHUNK = 128
CHUNKS = B_PER_W // IDX_CHUNK
GROUPS = B_PER_W // LANES


def _body(uid_hbm, iid_hbm, uemb_hbm, iemb_hbm, ub_hbm, ib_hbm, mean_hbm,
          out_hbm, uidx_v, iidx_v, u_rows, i_rows, ub_v, ib_v, out_v,
          mean_v, sem):
    wid = lax.axis_index("s") * NC + lax.axis_index("c")
    base_row = wid * CHUNKS
    base = wid * B_PER_W

    pltpu.sync_copy(uid_hbm.at[pl.ds(base_row, CHUNKS)], uidx_v)
    pltpu.sync_copy(iid_hbm.at[pl.ds(base_row, CHUNKS)], iidx_v)
    pltpu.sync_copy(mean_hbm, mean_v)

    copies = []
    for j in range(CHUNKS):
        sl = pl.ds(j * IDX_CHUNK, IDX_CHUNK)
        copies.append(pltpu.async_copy(
            uemb_hbm.at[uidx_v.at[j]], u_rows.at[sl], sem))
        copies.append(pltpu.async_copy(
            iemb_hbm.at[iidx_v.at[j]], i_rows.at[sl], sem))
        copies.append(pltpu.async_copy(
            ub_hbm.at[uidx_v.at[j]], ub_v.at[sl], sem))
        copies.append(pltpu.async_copy(
            ib_hbm.at[iidx_v.at[j]], ib_v.at[sl], sem))
    for c in copies:
        c.wait()

    H = EMBED_DIM // 2
    lane = lax.iota(jnp.int32, LANES)
    zeros16 = jnp.zeros((LANES,), jnp.float32)
    mean16 = mean_v[...]

    def step(s, carry):
        r0 = s * LANES
        acc = zeros16
        for r in range(LANES):
            u0 = u_rows[r0 + r, pl.ds(0, H)]
            u1 = u_rows[r0 + r, pl.ds(H, H)]
            i0 = i_rows[r0 + r, pl.ds(0, H)]
            i1 = i_rows[r0 + r, pl.ds(H, H)]
            p = u0 * i0 + u1 * i1
            acc = jnp.where(lane == r, jnp.sum(p), acc)
        out_v[pl.ds(r0, LANES)] = acc
        return carry

    lax.fori_loop(0, GROUPS, step, 0)

    for g in range(GROUPS):
        sl = pl.ds(g * LANES, LANES)
        out_v[sl] = out_v[sl] + ub_v[sl] + ib_v[sl] + mean16

    pltpu.sync_copy(out_v, out_hbm.at[pl.ds(base, B_PER_W)])


@jax.jit
def _run(u_id2d, i_id2d, user_emb, item_emb, ub_flat, ib_flat, mean16):
    mesh = plsc.VectorSubcoreMesh(core_axis_name="c", subcore_axis_name="s")
    call = pl.kernel(
        _body,
        out_type=jax.ShapeDtypeStruct((BATCH_SIZE,), jnp.float32),
        mesh=mesh,
        compiler_params=pltpu.CompilerParams(
            needs_layout_passes=False, use_tc_tiling_on_sc=False),
        scratch_types=[
            pltpu.VMEM((CHUNKS, IDX_CHUNK), jnp.int32),
            pltpu.VMEM((CHUNKS, IDX_CHUNK), jnp.int32),
            pltpu.VMEM((B_PER_W, EMBED_DIM), jnp.float32),
            pltpu.VMEM((B_PER_W, EMBED_DIM), jnp.float32),
            pltpu.VMEM((B_PER_W,), jnp.float32),
            pltpu.VMEM((B_PER_W,), jnp.float32),
            pltpu.VMEM((B_PER_W,), jnp.float32),
            pltpu.VMEM((LANES,), jnp.float32),
            pltpu.SemaphoreType.DMA,
        ],
    )
    return call(u_id2d, i_id2d, user_emb, item_emb, ub_flat, ib_flat, mean16)


def kernel(u_id, i_id, user_emb, item_emb, user_bias, item_bias, mean):
    u_id2d = u_id.astype(jnp.int32).reshape(NW * CHUNKS, IDX_CHUNK)
    i_id2d = i_id.astype(jnp.int32).reshape(NW * CHUNKS, IDX_CHUNK)
    ub_flat = user_bias.reshape(-1)
    ib_flat = item_bias.reshape(-1)
    mean16 = jnp.broadcast_to(mean.astype(jnp.float32).reshape(()), (LANES,))
    return _run(u_id2d, i_id2d, user_emb, item_emb, ub_flat, ib_flat, mean16)


# speedup vs baseline: 1.1747x; 1.0009x over previous
import jax
import jax.numpy as jnp
from jax import lax
from jax.experimental import pallas as pl
from jax.experimental.pallas import tpu as pltpu
from jax.experimental.pallas import tpu_sc as plsc

NUM_ROWS_TABLE = 1_000_000
EMBED_DIM = 32
BATCH_SIZE = 16384

NC = 2
NS = 16
LANES = 16
NW = NC * NS
B_PER_W = BATCH_SIZE // NW
IDX_CHUNK = 128
CHUNKS = B_PER_W // IDX_CHUNK
GROUPS = B_PER_W // LANES


def _body(uid_hbm, iid_hbm, uemb_hbm, iemb_hbm, ub_hbm, ib_hbm, mean_hbm,
          out_hbm, uidx_v, iidx_v, u_rows, i_rows, ub_v, ib_v, out_v,
          mean_v, sem):
    wid = lax.axis_index("s") * NC + lax.axis_index("c")
    base_row = wid * CHUNKS
    base = wid * B_PER_W

    pltpu.sync_copy(uid_hbm.at[pl.ds(base_row, CHUNKS)], uidx_v)
    pltpu.sync_copy(iid_hbm.at[pl.ds(base_row, CHUNKS)], iidx_v)
    pltpu.sync_copy(mean_hbm, mean_v)

    copies = []
    for j in range(CHUNKS):
        sl = pl.ds(j * IDX_CHUNK, IDX_CHUNK)
        copies.append(pltpu.async_copy(
            uemb_hbm.at[uidx_v.at[j]], u_rows.at[sl], sem))
        copies.append(pltpu.async_copy(
            iemb_hbm.at[iidx_v.at[j]], i_rows.at[sl], sem))
        copies.append(pltpu.async_copy(
            ub_hbm.at[uidx_v.at[j]], ub_v.at[sl], sem))
        copies.append(pltpu.async_copy(
            ib_hbm.at[iidx_v.at[j]], ib_v.at[sl], sem))
    for c in copies:
        c.wait()

    H = EMBED_DIM // 2
    lane = lax.iota(jnp.int32, LANES)
    zeros16 = jnp.zeros((LANES,), jnp.float32)
    mean16 = mean_v[...]

    def step(s, carry):
        r0 = s * LANES
        acc = zeros16
        for r in range(LANES):
            u0 = u_rows[r0 + r, pl.ds(0, H)]
            u1 = u_rows[r0 + r, pl.ds(H, H)]
            i0 = i_rows[r0 + r, pl.ds(0, H)]
            i1 = i_rows[r0 + r, pl.ds(H, H)]
            p = u0 * i0 + u1 * i1
            acc = jnp.where(lane == r, jnp.sum(p), acc)
        out_v[pl.ds(r0, LANES)] = acc
        return carry

    lax.fori_loop(0, GROUPS, step, 0)

    for g in range(GROUPS):
        sl = pl.ds(g * LANES, LANES)
        out_v[sl] = out_v[sl] + ub_v[sl] + ib_v[sl] + mean16

    pltpu.sync_copy(out_v, out_hbm.at[pl.ds(base, B_PER_W)])


@jax.jit
def _run(u_id2d, i_id2d, user_emb, item_emb, ub_flat, ib_flat, mean16):
    mesh = plsc.VectorSubcoreMesh(core_axis_name="c", subcore_axis_name="s")
    call = pl.kernel(
        _body,
        out_type=jax.ShapeDtypeStruct((BATCH_SIZE,), jnp.float32),
        mesh=mesh,
        compiler_params=pltpu.CompilerParams(
            needs_layout_passes=False, use_tc_tiling_on_sc=False),
        scratch_types=[
            pltpu.VMEM((CHUNKS, IDX_CHUNK), jnp.int32),
            pltpu.VMEM((CHUNKS, IDX_CHUNK), jnp.int32),
            pltpu.VMEM((B_PER_W, EMBED_DIM), jnp.float32),
            pltpu.VMEM((B_PER_W, EMBED_DIM), jnp.float32),
            pltpu.VMEM((B_PER_W,), jnp.float32),
            pltpu.VMEM((B_PER_W,), jnp.float32),
            pltpu.VMEM((B_PER_W,), jnp.float32),
            pltpu.VMEM((LANES,), jnp.float32),
            pltpu.SemaphoreType.DMA,
        ],
    )
    return call(u_id2d, i_id2d, user_emb, item_emb, ub_flat, ib_flat, mean16)


def kernel(u_id, i_id, user_emb, item_emb, user_bias, item_bias, mean):
    u_id2d = u_id.astype(jnp.int32).reshape(NW * CHUNKS, IDX_CHUNK)
    i_id2d = i_id.astype(jnp.int32).reshape(NW * CHUNKS, IDX_CHUNK)
    ub_flat = jnp.sum(user_bias, axis=1)
    ib_flat = jnp.sum(item_bias, axis=1)
    mean16 = jnp.broadcast_to(mean.astype(jnp.float32).reshape(()), (LANES,))
    return _run(u_id2d, i_id2d, user_emb, item_emb, ub_flat, ib_flat, mean16)


# speedup vs baseline: 1.1751x; 1.0003x over previous
import jax
import jax.numpy as jnp
from jax import lax
from jax.experimental import pallas as pl
from jax.experimental.pallas import tpu as pltpu
from jax.experimental.pallas import tpu_sc as plsc

NUM_ROWS_TABLE = 1_000_000
EMBED_DIM = 32
BATCH_SIZE = 16384

NC = 2
NS = 16
LANES = 16
NW = NC * NS
B_PER_W = BATCH_SIZE // NW
IDX_CHUNK = 128
CHUNKS = B_PER_W // IDX_CHUNK
GROUPS = B_PER_W // LANES


def _body(uid_hbm, iid_hbm, uemb_hbm, iemb_hbm, ub_hbm, ib_hbm, mean_hbm,
          out_hbm, uidx_v, iidx_v, u_rows, i_rows, ub_v, ib_v, out_v,
          mean_v, sem):
    wid = lax.axis_index("s") * NC + lax.axis_index("c")
    base_row = wid * CHUNKS
    base = wid * B_PER_W

    pltpu.sync_copy(uid_hbm.at[pl.ds(base_row, CHUNKS)], uidx_v)
    pltpu.sync_copy(iid_hbm.at[pl.ds(base_row, CHUNKS)], iidx_v)
    pltpu.sync_copy(mean_hbm, mean_v)

    copies = []
    for j in range(CHUNKS):
        sl = pl.ds(j * IDX_CHUNK, IDX_CHUNK)
        copies.append(pltpu.async_copy(
            uemb_hbm.at[uidx_v.at[j]], u_rows.at[sl], sem))
        copies.append(pltpu.async_copy(
            iemb_hbm.at[iidx_v.at[j]], i_rows.at[sl], sem))
        copies.append(pltpu.async_copy(
            ub_hbm.at[uidx_v.at[j]], ub_v.at[sl], sem))
        copies.append(pltpu.async_copy(
            ib_hbm.at[iidx_v.at[j]], ib_v.at[sl], sem))
    for c in copies:
        c.wait()

    H = EMBED_DIM // 2
    lane = lax.iota(jnp.int32, LANES)
    zeros16 = jnp.zeros((LANES,), jnp.float32)
    mean16 = mean_v[...]

    def step(s, carry):
        r0 = s * LANES
        acc = zeros16
        for r in range(LANES):
            u0 = u_rows[r0 + r, pl.ds(0, H)]
            u1 = u_rows[r0 + r, pl.ds(H, H)]
            i0 = i_rows[r0 + r, pl.ds(0, H)]
            i1 = i_rows[r0 + r, pl.ds(H, H)]
            p = u0 * i0 + u1 * i1
            acc = jnp.where(lane == r, jnp.sum(p), acc)
        out_v[pl.ds(r0, LANES)] = acc
        return carry

    lax.fori_loop(0, GROUPS, step, 0)

    for g in range(GROUPS):
        sl = pl.ds(g * LANES, LANES)
        out_v[sl] = out_v[sl] + ub_v[sl] + ib_v[sl] + mean16

    pltpu.sync_copy(out_v, out_hbm.at[pl.ds(base, B_PER_W)])


@jax.jit
def _run(u_id2d, i_id2d, user_emb, item_emb, ub_flat, ib_flat, mean16):
    mesh = plsc.VectorSubcoreMesh(core_axis_name="c", subcore_axis_name="s")
    call = pl.kernel(
        _body,
        out_type=jax.ShapeDtypeStruct((BATCH_SIZE,), jnp.float32),
        mesh=mesh,
        compiler_params=pltpu.CompilerParams(
            needs_layout_passes=False, use_tc_tiling_on_sc=False),
        scratch_types=[
            pltpu.VMEM((CHUNKS, IDX_CHUNK), jnp.int32),
            pltpu.VMEM((CHUNKS, IDX_CHUNK), jnp.int32),
            pltpu.VMEM((B_PER_W, EMBED_DIM), jnp.float32),
            pltpu.VMEM((B_PER_W, EMBED_DIM), jnp.float32),
            pltpu.VMEM((B_PER_W,), jnp.float32),
            pltpu.VMEM((B_PER_W,), jnp.float32),
            pltpu.VMEM((B_PER_W,), jnp.float32),
            pltpu.VMEM((LANES,), jnp.float32),
            pltpu.SemaphoreType.DMA,
        ],
    )
    return call(u_id2d, i_id2d, user_emb, item_emb, ub_flat, ib_flat, mean16)


def kernel(u_id, i_id, user_emb, item_emb, user_bias, item_bias, mean):
    u_id2d = u_id.astype(jnp.int32).reshape(NW * CHUNKS, IDX_CHUNK)
    i_id2d = i_id.astype(jnp.int32).reshape(NW * CHUNKS, IDX_CHUNK)
    ub_flat = user_bias.reshape(-1)
    ib_flat = item_bias.reshape(-1)
    mean16 = jnp.broadcast_to(mean.astype(jnp.float32).reshape(()), (LANES,))
    return _run(u_id2d, i_id2d, user_emb, item_emb, ub_flat, ib_flat, mean16)
